# baseline (device time: 41696 ns/iter reference)
import jax
import jax.numpy as jnp
from jax import lax
from jax.experimental import pallas as pl
from jax.experimental.pallas import tpu as pltpu

N_DEV = 32
B, SQ, SKV, H_LOC, DH = 2, 256, 256, 4, 64
D_MODEL = 512
D_HEADS = H_LOC * DH
ROWS = B * SQ
CHUNK = ROWS // N_DEV
CHUNKS_PER_B = SQ // CHUNK


def kernel(x, Wq, K_ext, V_ext, Wo):
    me = lax.axis_index("i")
    wq_s = lax.dynamic_slice(
        Wq, (0, me * D_HEADS), (Wq.shape[0], D_HEADS)).astype(jnp.bfloat16)
    wo_s = lax.dynamic_slice(
        Wo, (me * D_HEADS, 0), (D_HEADS, Wo.shape[1])).astype(jnp.bfloat16)
    kt = jnp.transpose(K_ext, (0, 2, 1, 3)).astype(jnp.bfloat16)
    vt = jnp.transpose(V_ext, (0, 2, 1, 3)).astype(jnp.bfloat16)
    xb = x.astype(jnp.bfloat16)

    def body(x_ref, wq_ref, kt_ref, vt_ref, wo_ref, out_ref,
             part_ref, rs_ref,
             send_a, recv_a, send_b, recv_b):
        my = lax.axis_index("i")

        barrier_sem = pltpu.get_barrier_semaphore()
        for d in range(1, N_DEV):
            pl.semaphore_signal(
                barrier_sem, inc=1,
                device_id=((my + d) % N_DEV,),
                device_id_type=pl.DeviceIdType.MESH,
            )

        qi = lax.broadcasted_iota(jnp.int32, (SQ, SKV), 0)
        ki = lax.broadcasted_iota(jnp.int32, (SQ, SKV), 1)
        mask = (jnp.abs(qi - ki) <= 128) | (ki < 32) | (qi < 32)

        a_rdmas = []
        for d in range(1, N_DEV):
            j = (my + d) % N_DEV
            r = pltpu.make_async_remote_copy(
                src_ref=part_ref.at[pl.ds(j * CHUNK, CHUNK)],
                dst_ref=rs_ref.at[d - 1],
                send_sem=send_a.at[d - 1],
                recv_sem=recv_a.at[d - 1],
                device_id=(j,),
                device_id_type=pl.DeviceIdType.MESH,
            )
            a_rdmas.append((r, j))

        wq_bf = wq_ref[...]
        wo_bf = wo_ref[...]

        for b in range(B):
            q_b = jnp.dot(x_ref[b], wq_bf,
                          preferred_element_type=jnp.float32)
            q_b = q_b.astype(jnp.bfloat16)
            acc = None
            for h in range(H_LOC):
                q_bh = q_b[:, h * DH:(h + 1) * DH]
                k_bh = kt_ref[b, h]
                s = lax.dot_general(
                    q_bh, k_bh, (((1,), (1,)), ((), ())),
                    preferred_element_type=jnp.float32) * 0.125
                s = jnp.where(mask, s, -1e9)
                m = jnp.max(s, axis=1, keepdims=True)
                w = jnp.exp(s - m)
                w = (w / jnp.sum(w, axis=1, keepdims=True)).astype(jnp.bfloat16)
                ctx = jnp.dot(w, vt_ref[b, h],
                              preferred_element_type=jnp.float32)
                contrib = jnp.dot(ctx.astype(jnp.bfloat16),
                                  wo_bf[h * DH:(h + 1) * DH, :],
                                  preferred_element_type=jnp.float32)
                acc = contrib if acc is None else acc + contrib
            part_ref[b * SQ:(b + 1) * SQ, :] = acc.astype(jnp.bfloat16)

            if b == 0:
                pl.semaphore_wait(barrier_sem, N_DEV - 1)
            lo, hi = b * CHUNKS_PER_B, (b + 1) * CHUNKS_PER_B
            for r, j in a_rdmas:
                @pl.when((j >= lo) & (j < hi))
                def _(r=r):
                    r.start()

        red = part_ref[pl.ds(my * CHUNK, CHUNK)].astype(jnp.float32)
        for d, (r, _) in enumerate(a_rdmas):
            r.wait_recv()
            red = red + rs_ref[d].astype(jnp.float32)
        out_ref[pl.ds(my * CHUNK, CHUNK)] = red

        b_rdmas = []
        for d in range(1, N_DEV):
            j = (my + d) % N_DEV
            r = pltpu.make_async_remote_copy(
                src_ref=out_ref.at[pl.ds(my * CHUNK, CHUNK)],
                dst_ref=out_ref.at[pl.ds(my * CHUNK, CHUNK)],
                send_sem=send_b.at[d - 1],
                recv_sem=recv_b.at[d - 1],
                device_id=(j,),
                device_id_type=pl.DeviceIdType.MESH,
            )
            r.start()
            b_rdmas.append(r)
        for r in b_rdmas:
            r.wait_recv()
        for r, _ in a_rdmas:
            r.wait_send()
        for r in b_rdmas:
            r.wait_send()

    out2d = pl.pallas_call(
        body,
        out_shape=jax.ShapeDtypeStruct((ROWS, D_MODEL), jnp.float32),
        in_specs=[pl.BlockSpec(memory_space=pltpu.MemorySpace.VMEM)] * 5,
        out_specs=pl.BlockSpec(memory_space=pltpu.MemorySpace.VMEM),
        scratch_shapes=[
            pltpu.VMEM((ROWS, D_MODEL), jnp.bfloat16),
            pltpu.VMEM((N_DEV - 1, CHUNK, D_MODEL), jnp.bfloat16),
            pltpu.SemaphoreType.DMA((N_DEV - 1,)),
            pltpu.SemaphoreType.DMA((N_DEV - 1,)),
            pltpu.SemaphoreType.DMA((N_DEV - 1,)),
            pltpu.SemaphoreType.DMA((N_DEV - 1,)),
        ],
        compiler_params=pltpu.CompilerParams(collective_id=0),
    )(xb, wq_s, kt, vt, wo_s)
    return out2d.reshape(B, SQ, D_MODEL)


# device time: 12140 ns/iter; 3.4346x vs baseline; 3.4346x over previous
import jax
import jax.numpy as jnp
from jax import lax
from jax.experimental import pallas as pl
from jax.experimental.pallas import tpu as pltpu

N_DEV = 32
B, SQ, SKV, H_LOC, DH = 2, 256, 256, 4, 64
D_MODEL = 512
D_HEADS = H_LOC * DH
ROWS = B * SQ
CHUNK = ROWS // N_DEV
CHUNKS_PER_B = SQ // CHUNK


def kernel(x, Wq, K_ext, V_ext, Wo):
    me = lax.axis_index("i")
    wq_s = lax.dynamic_slice(
        Wq, (0, me * D_HEADS), (Wq.shape[0], D_HEADS)).astype(jnp.bfloat16)
    wo_s = lax.dynamic_slice(
        Wo, (me * D_HEADS, 0), (D_HEADS, Wo.shape[1])).astype(jnp.bfloat16)
    kt = jnp.transpose(K_ext, (0, 2, 1, 3)).astype(jnp.bfloat16)
    vt = jnp.transpose(V_ext, (0, 2, 1, 3)).astype(jnp.bfloat16)
    xb = x.astype(jnp.bfloat16)

    def body(x_ref, wq_ref, kt_ref, vt_ref, wo_ref, out_ref,
             part_ref, rs_ref, ag_ref,
             send_a, recv_a, send_b, recv_b):
        my = lax.axis_index("i")

        barrier_sem = pltpu.get_barrier_semaphore()
        for d in range(1, N_DEV):
            pl.semaphore_signal(
                barrier_sem, inc=1,
                device_id=((my + d) % N_DEV,),
                device_id_type=pl.DeviceIdType.MESH,
            )

        qi = lax.broadcasted_iota(jnp.int32, (SQ, SKV), 0)
        ki = lax.broadcasted_iota(jnp.int32, (SQ, SKV), 1)
        mask = (jnp.abs(qi - ki) <= 128) | (ki < 32) | (qi < 32)

        a_rdmas = []
        for d in range(1, N_DEV):
            j = (my + d) % N_DEV
            r = pltpu.make_async_remote_copy(
                src_ref=part_ref.at[pl.ds(j * CHUNK, CHUNK)],
                dst_ref=rs_ref.at[d - 1],
                send_sem=send_a.at[d - 1],
                recv_sem=recv_a.at[d - 1],
                device_id=(j,),
                device_id_type=pl.DeviceIdType.MESH,
            )
            a_rdmas.append((r, j))

        wq_bf = wq_ref[...]
        wo_bf = wo_ref[...]

        for b in range(B):
            q_b = jnp.dot(x_ref[b], wq_bf,
                          preferred_element_type=jnp.float32)
            q_b = q_b.astype(jnp.bfloat16)
            acc = None
            for h in range(H_LOC):
                q_bh = q_b[:, h * DH:(h + 1) * DH]
                k_bh = kt_ref[b, h]
                s = lax.dot_general(
                    q_bh, k_bh, (((1,), (1,)), ((), ())),
                    preferred_element_type=jnp.float32) * 0.125
                s = jnp.where(mask, s, -1e9)
                m = jnp.max(s, axis=1, keepdims=True)
                w = jnp.exp(s - m)
                w = (w / jnp.sum(w, axis=1, keepdims=True)).astype(jnp.bfloat16)
                ctx = jnp.dot(w, vt_ref[b, h],
                              preferred_element_type=jnp.float32)
                contrib = jnp.dot(ctx.astype(jnp.bfloat16),
                                  wo_bf[h * DH:(h + 1) * DH, :],
                                  preferred_element_type=jnp.float32)
                acc = contrib if acc is None else acc + contrib
            part_ref[b * SQ:(b + 1) * SQ, :] = acc.astype(jnp.bfloat16)

            if b == 0:
                pl.semaphore_wait(barrier_sem, N_DEV - 1)
            lo, hi = b * CHUNKS_PER_B, (b + 1) * CHUNKS_PER_B
            for r, j in a_rdmas:
                @pl.when((j >= lo) & (j < hi))
                def _(r=r):
                    r.start()

        red = part_ref[pl.ds(my * CHUNK, CHUNK)].astype(jnp.float32)
        for d, (r, _) in enumerate(a_rdmas):
            r.wait_recv()
            red = red + rs_ref[d].astype(jnp.float32)
        ag_ref[my] = red.astype(jnp.bfloat16)

        b_rdmas = []
        for d in range(1, N_DEV):
            j = (my + d) % N_DEV
            r = pltpu.make_async_remote_copy(
                src_ref=ag_ref.at[my],
                dst_ref=ag_ref.at[my],
                send_sem=send_b.at[d - 1],
                recv_sem=recv_b.at[d - 1],
                device_id=(j,),
                device_id_type=pl.DeviceIdType.MESH,
            )
            r.start()
            b_rdmas.append(r)
        for r in b_rdmas:
            r.wait_recv()
        out_ref[...] = ag_ref[...].reshape(ROWS, D_MODEL).astype(jnp.float32)
        for r, _ in a_rdmas:
            r.wait_send()
        for r in b_rdmas:
            r.wait_send()

    out2d = pl.pallas_call(
        body,
        out_shape=jax.ShapeDtypeStruct((ROWS, D_MODEL), jnp.float32),
        in_specs=[pl.BlockSpec(memory_space=pltpu.MemorySpace.VMEM)] * 5,
        out_specs=pl.BlockSpec(memory_space=pltpu.MemorySpace.VMEM),
        scratch_shapes=[
            pltpu.VMEM((ROWS, D_MODEL), jnp.bfloat16),
            pltpu.VMEM((N_DEV - 1, CHUNK, D_MODEL), jnp.bfloat16),
            pltpu.VMEM((N_DEV, CHUNK, D_MODEL), jnp.bfloat16),
            pltpu.SemaphoreType.DMA((N_DEV - 1,)),
            pltpu.SemaphoreType.DMA((N_DEV - 1,)),
            pltpu.SemaphoreType.DMA((N_DEV - 1,)),
            pltpu.SemaphoreType.DMA((N_DEV - 1,)),
        ],
        compiler_params=pltpu.CompilerParams(collective_id=0),
    )(xb, wq_s, kt, vt, wo_s)
    return out2d.reshape(B, SQ, D_MODEL)
